# baseline (speedup 1.0000x reference)
import jax
import jax.numpy as jnp
from jax.experimental import pallas as pl
from jax.experimental.pallas import tpu as pltpu

B = 8
T = 512
T_A = 512
H = 256
HEADS = 4
HH = HEADS * H

_NEG = -1e30


def _lrelu(x):
    return jnp.maximum(x, 0.2 * x)


def _fused_kernel(rm_ref, rl_ref, rr_ref, ra_ref,
                  Wm_ref, bm_ref, Wl_ref, bl_ref, Wr_ref, br_ref, Wa_ref, ba_ref,
                  W0_ref, as0_ref, ad0_ref, b0_ref,
                  W1_ref, as1_ref, ad1_ref, b1_ref,
                  W2_ref, as2_ref, ad2_ref, b2_ref,
                  g_ref, be_ref,
                  sums_ref):
    f32 = jnp.float32

    def mm(a, w):
        return jnp.dot(a, w, preferred_element_type=f32)

    xM = mm(rm_ref[0], Wm_ref[...]) + bm_ref[...]
    xL = mm(rl_ref[0], Wl_ref[...]) + bl_ref[...]
    xR = mm(rr_ref[0], Wr_ref[...]) + br_ref[...]
    xA = mm(ra_ref[0], Wa_ref[...]) + ba_ref[...]
    X = jnp.concatenate([xM, xL, xR, xA], axis=0)

    t0_mask = jax.lax.broadcasted_iota(jnp.int32, (T, HEADS), 0) == 0
    hsl = [slice(k * H, (k + 1) * H) for k in range(HEADS)]

    def gat(X, W_ref, as_ref, ad_ref, b_ref, concat):
        W = W_ref[...]
        att_s = as_ref[...]
        att_d = ad_ref[...]

        Hc = mm(X, W)
        a_s = jnp.concatenate(
            [jnp.sum(Hc[:, hsl[k]] * att_s[k:k + 1, :], axis=1, keepdims=True)
             for k in range(HEADS)], axis=1)
        a_d = jnp.concatenate(
            [jnp.sum(Hc[:, hsl[k]] * att_d[k:k + 1, :], axis=1, keepdims=True)
             for k in range(HEADS)], axis=1)

        hM, hL, hR, hA = Hc[0:T], Hc[T:2 * T], Hc[2 * T:3 * T], Hc[3 * T:4 * T]
        asM, asL, asR, asA = a_s[0:T], a_s[T:2 * T], a_s[2 * T:3 * T], a_s[3 * T:4 * T]
        adM, adL, adR, adA = a_d[0:T], a_d[T:2 * T], a_d[2 * T:3 * T], a_d[3 * T:4 * T]

        def combine(ws_hs):
            return [sum(w[:, k:k + 1] * h[:, hsl[k]] for w, h in ws_hs)
                    for k in range(HEADS)]

        def face(h_j, as_j, ad_j, h_o1, as_o1, h_o2, as_o2):
            a1 = _lrelu(as_o1 + ad_j)
            a2 = _lrelu(as_o2 + ad_j)
            at = _lrelu(jnp.roll(as_j, 1, axis=0) + ad_j)
            at = jnp.where(t0_mask, _NEG, at)
            a0 = _lrelu(as_j + ad_j)
            m = jnp.maximum(jnp.maximum(a1, a2), jnp.maximum(at, a0))
            e1 = jnp.exp(a1 - m)
            e2 = jnp.exp(a2 - m)
            et = jnp.exp(at - m)
            e0 = jnp.exp(a0 - m)
            r = 1.0 / (e1 + e2 + et + e0 + 1e-16)
            h_prev = jnp.roll(h_j, 1, axis=0)
            return combine([(e1 * r, h_o1), (e2 * r, h_o2),
                            (et * r, h_prev), (e0 * r, h_j)])

        oM = face(hM, asM, adM, hL, asL, hR, asR)
        oL = face(hL, asL, adL, hM, asM, hR, asR)
        oR = face(hR, asR, adR, hM, asM, hL, asL)

        aL = _lrelu(asL + adA)
        aR = _lrelu(asR + adA)
        a0 = _lrelu(asA + adA)
        m = jnp.maximum(jnp.maximum(aL, aR), a0)
        eL = jnp.exp(aL - m)
        eR = jnp.exp(aR - m)
        e0 = jnp.exp(a0 - m)
        r = 1.0 / (eL + eR + e0 + 1e-16)
        oA = combine([(eL * r, hL), (eR * r, hR), (e0 * r, hA)])

        bias = b_ref[...]
        if concat:
            return jnp.concatenate(
                [jnp.concatenate(o, axis=1) + bias for o in (oM, oL, oR, oA)], axis=0)
        return jnp.concatenate(
            [0.25 * (o[0] + o[1] + o[2] + o[3]) + bias for o in (oM, oL, oR, oA)], axis=0)

    X = gat(X, W0_ref, as0_ref, ad0_ref, b0_ref, True)
    X = gat(X, W1_ref, as1_ref, ad1_ref, b1_ref, True)
    X = gat(X, W2_ref, as2_ref, ad2_ref, b2_ref, False)

    gamma = g_ref[...]
    beta = be_ref[...]
    mu = jnp.mean(X, axis=1, keepdims=True)
    var = jnp.mean((X - mu) ** 2, axis=1, keepdims=True)
    xn = (X - mu) * jax.lax.rsqrt(var + 1e-5) * gamma + beta
    for r_idx in range(4):
        sums_ref[0, r_idx, :] = jnp.sum(xn[r_idx * T:(r_idx + 1) * T], axis=0)


def kernel(region_mouth, region_left_eye, region_right_eye, audio_embeddings,
           W_mouth, b_mouth, W_leye, b_leye, W_reye, b_reye, W_audio, b_audio,
           W0, att_src0, att_dst0, bias0, W1, att_src1, att_dst1, bias1,
           W2, att_src2, att_dst2, bias2, ln_gamma, ln_beta):
    f32 = jnp.float32

    row = lambda v: v.reshape(1, -1)

    full = lambda shape: pl.BlockSpec(shape, lambda b: (0,) * len(shape))
    per_b = lambda shape: pl.BlockSpec(shape, lambda b: (b, 0, 0))

    sums = pl.pallas_call(
        _fused_kernel,
        grid=(B,),
        in_specs=[
            per_b((1, T, 512)), per_b((1, T, 256)), per_b((1, T, 256)), per_b((1, T_A, 128)),
            full((512, H)), full((1, H)),
            full((256, H)), full((1, H)),
            full((256, H)), full((1, H)),
            full((128, H)), full((1, H)),
            full((H, HH)), full((HEADS, H)), full((HEADS, H)), full((1, HH)),
            full((HH, HH)), full((HEADS, H)), full((HEADS, H)), full((1, HH)),
            full((HH, HH)), full((HEADS, H)), full((HEADS, H)), full((1, H)),
            full((1, H)), full((1, H)),
        ],
        out_specs=pl.BlockSpec((1, 4, H), lambda b: (b, 0, 0)),
        out_shape=jax.ShapeDtypeStruct((B, 4, H), f32),
        compiler_params=pltpu.CompilerParams(
            dimension_semantics=("arbitrary",),
        ),
    )(region_mouth, region_left_eye, region_right_eye, audio_embeddings,
      W_mouth, row(b_mouth), W_leye, row(b_leye), W_reye, row(b_reye),
      W_audio, row(b_audio),
      W0, att_src0, att_dst0, row(bias0),
      W1, att_src1, att_dst1, row(bias1),
      W2, att_src2, att_dst2, row(bias2),
      row(ln_gamma), row(ln_beta))

    s2 = sums.reshape(2, 4, 4, H)
    out = s2.sum(axis=1).transpose(1, 0, 2).reshape(B, H)
    return out / 2048.0


# speedup vs baseline: 1.0695x; 1.0695x over previous
import jax
import jax.numpy as jnp
from jax.experimental import pallas as pl
from jax.experimental.pallas import tpu as pltpu

B = 8
T = 512
T_A = 512
H = 256
HEADS = 4
HH = HEADS * H

_NEG = -1e30


def _lrelu(x):
    return jnp.maximum(x, 0.2 * x)


def _fused_kernel(rm_ref, rl_ref, rr_ref, ra_ref,
                  Wm_ref, bm_ref, Wl_ref, bl_ref, Wr_ref, br_ref, Wa_ref, ba_ref,
                  W0_ref, b0_ref,
                  W1_ref, b1_ref,
                  W2_ref, b2_ref,
                  g_ref, be_ref,
                  sums_ref):
    f32 = jnp.float32

    def mm(a, w):
        return jnp.dot(a, w, preferred_element_type=f32)

    xM = mm(rm_ref[0], Wm_ref[...]) + bm_ref[...]
    xL = mm(rl_ref[0], Wl_ref[...]) + bl_ref[...]
    xR = mm(rr_ref[0], Wr_ref[...]) + br_ref[...]
    xA = mm(ra_ref[0], Wa_ref[...]) + ba_ref[...]

    t0_mask = jax.lax.broadcasted_iota(jnp.int32, (T, HEADS), 0) == 0
    hsl = [slice(k * H, (k + 1) * H) for k in range(HEADS)]

    def gat(xM, xL, xR, xA, W_ref, b_ref, concat):
        W = W_ref[...]

        def proj(x):
            o = mm(x, W)
            return o[:, 0:HH], o[:, HH:HH + HEADS], o[:, HH + HEADS:HH + 2 * HEADS]

        hM, asM, adM = proj(xM)
        hL, asL, adL = proj(xL)
        hR, asR, adR = proj(xR)
        hA, asA, adA = proj(xA)

        def combine(ws_hs):
            return [sum(w[:, k:k + 1] * h[:, hsl[k]] for w, h in ws_hs)
                    for k in range(HEADS)]

        def face(h_j, as_j, ad_j, h_o1, as_o1, h_o2, as_o2):
            a1 = _lrelu(as_o1 + ad_j)
            a2 = _lrelu(as_o2 + ad_j)
            at = _lrelu(jnp.roll(as_j, 1, axis=0) + ad_j)
            at = jnp.where(t0_mask, _NEG, at)
            a0 = _lrelu(as_j + ad_j)
            m = jnp.maximum(jnp.maximum(a1, a2), jnp.maximum(at, a0))
            e1 = jnp.exp(a1 - m)
            e2 = jnp.exp(a2 - m)
            et = jnp.exp(at - m)
            e0 = jnp.exp(a0 - m)
            r = 1.0 / (e1 + e2 + et + e0 + 1e-16)
            h_prev = jnp.roll(h_j, 1, axis=0)
            return combine([(e1 * r, h_o1), (e2 * r, h_o2),
                            (et * r, h_prev), (e0 * r, h_j)])

        oM = face(hM, asM, adM, hL, asL, hR, asR)
        oL = face(hL, asL, adL, hM, asM, hR, asR)
        oR = face(hR, asR, adR, hM, asM, hL, asL)

        aL = _lrelu(asL + adA)
        aR = _lrelu(asR + adA)
        a0 = _lrelu(asA + adA)
        m = jnp.maximum(jnp.maximum(aL, aR), a0)
        eL = jnp.exp(aL - m)
        eR = jnp.exp(aR - m)
        e0 = jnp.exp(a0 - m)
        r = 1.0 / (eL + eR + e0 + 1e-16)
        oA = combine([(eL * r, hL), (eR * r, hR), (e0 * r, hA)])

        bias = b_ref[...]
        if concat:
            return tuple(jnp.concatenate(o, axis=1) + bias for o in (oM, oL, oR, oA))
        return tuple(0.25 * (o[0] + o[1] + o[2] + o[3]) + bias for o in (oM, oL, oR, oA))

    xM, xL, xR, xA = gat(xM, xL, xR, xA, W0_ref, b0_ref, True)
    xM, xL, xR, xA = gat(xM, xL, xR, xA, W1_ref, b1_ref, True)
    xM, xL, xR, xA = gat(xM, xL, xR, xA, W2_ref, b2_ref, False)

    gamma = g_ref[...]
    beta = be_ref[...]
    for r_idx, x in enumerate((xM, xL, xR, xA)):
        mu = jnp.mean(x, axis=1, keepdims=True)
        var = jnp.mean((x - mu) ** 2, axis=1, keepdims=True)
        xn = (x - mu) * jax.lax.rsqrt(var + 1e-5) * gamma + beta
        sums_ref[0, r_idx, :] = jnp.sum(xn, axis=0)


def kernel(region_mouth, region_left_eye, region_right_eye, audio_embeddings,
           W_mouth, b_mouth, W_leye, b_leye, W_reye, b_reye, W_audio, b_audio,
           W0, att_src0, att_dst0, bias0, W1, att_src1, att_dst1, bias1,
           W2, att_src2, att_dst2, bias2, ln_gamma, ln_beta):
    f32 = jnp.float32

    eye4 = jnp.eye(HEADS, dtype=f32)

    def blockdiag(att):
        return (att[:, :, None] * eye4[:, None, :]).reshape(HH, HEADS)

    def augment(Wl, att_s, att_d):
        return jnp.concatenate(
            [Wl, Wl @ blockdiag(att_s), Wl @ blockdiag(att_d)], axis=1)

    W0a = augment(W0, att_src0, att_dst0)
    W1a = augment(W1, att_src1, att_dst1)
    W2a = augment(W2, att_src2, att_dst2)
    HA = HH + 2 * HEADS

    row = lambda v: v.reshape(1, -1)

    full = lambda shape: pl.BlockSpec(shape, lambda b: (0,) * len(shape))
    per_b = lambda shape: pl.BlockSpec(shape, lambda b: (b, 0, 0))

    sums = pl.pallas_call(
        _fused_kernel,
        grid=(B,),
        in_specs=[
            per_b((1, T, 512)), per_b((1, T, 256)), per_b((1, T, 256)), per_b((1, T_A, 128)),
            full((512, H)), full((1, H)),
            full((256, H)), full((1, H)),
            full((256, H)), full((1, H)),
            full((128, H)), full((1, H)),
            full((H, HA)), full((1, HH)),
            full((HH, HA)), full((1, HH)),
            full((HH, HA)), full((1, H)),
            full((1, H)), full((1, H)),
        ],
        out_specs=pl.BlockSpec((1, 4, H), lambda b: (b, 0, 0)),
        out_shape=jax.ShapeDtypeStruct((B, 4, H), f32),
        compiler_params=pltpu.CompilerParams(
            dimension_semantics=("arbitrary",),
        ),
    )(region_mouth, region_left_eye, region_right_eye, audio_embeddings,
      W_mouth, row(b_mouth), W_leye, row(b_leye), W_reye, row(b_reye),
      W_audio, row(b_audio),
      W0a, row(bias0),
      W1a, row(bias1),
      W2a, row(bias2),
      row(ln_gamma), row(ln_beta))

    s2 = sums.reshape(2, 4, 4, H)
    out = s2.sum(axis=1).transpose(1, 0, 2).reshape(B, H)
    return out / 2048.0


# speedup vs baseline: 1.1568x; 1.0816x over previous
import jax
import jax.numpy as jnp
from jax.experimental import pallas as pl
from jax.experimental.pallas import tpu as pltpu

B = 8
T = 512
T_A = 512
H = 256
HEADS = 4
HH = HEADS * H

_NEG = -1e30


def _lrelu(x):
    return jnp.maximum(x, 0.2 * x)


def _fused_kernel(rm_ref, rl_ref, rr_ref, ra_ref,
                  Wm_ref, bm_ref, Wl_ref, bl_ref, Wr_ref, br_ref, Wa_ref, ba_ref,
                  W0_ref, as0_ref, ad0_ref, b0_ref,
                  W1_ref, as1_ref, ad1_ref, b1_ref,
                  W2_ref, as2_ref, ad2_ref, b2_ref,
                  g_ref, be_ref,
                  sums_ref):
    f32 = jnp.float32

    def mm(a, w):
        return jnp.dot(a, w, preferred_element_type=f32)

    xM = mm(rm_ref[0], Wm_ref[...]) + bm_ref[...]
    xL = mm(rl_ref[0], Wl_ref[...]) + bl_ref[...]
    xR = mm(rr_ref[0], Wr_ref[...]) + br_ref[...]
    xA = mm(ra_ref[0], Wa_ref[...]) + ba_ref[...]

    t0_mask = jax.lax.broadcasted_iota(jnp.int32, (T, HEADS), 0) == 0
    hsl = [slice(k * H, (k + 1) * H) for k in range(HEADS)]

    def gat(xM, xL, xR, xA, W_ref, as_ref, ad_ref, b_ref, concat):
        W = W_ref[...]
        att_s = as_ref[...]
        att_d = ad_ref[...]

        def logits(h):
            a_s = jnp.concatenate(
                [jnp.sum(h[:, hsl[k]] * att_s[k:k + 1, :], axis=1, keepdims=True)
                 for k in range(HEADS)], axis=1)
            a_d = jnp.concatenate(
                [jnp.sum(h[:, hsl[k]] * att_d[k:k + 1, :], axis=1, keepdims=True)
                 for k in range(HEADS)], axis=1)
            return a_s, a_d

        hM, hL, hR, hA = mm(xM, W), mm(xL, W), mm(xR, W), mm(xA, W)
        asM, adM = logits(hM)
        asL, adL = logits(hL)
        asR, adR = logits(hR)
        asA, adA = logits(hA)

        def combine(ws_hs):
            return [sum(w[:, k:k + 1] * h[:, hsl[k]] for w, h in ws_hs)
                    for k in range(HEADS)]

        def face(h_j, as_j, ad_j, h_o1, as_o1, h_o2, as_o2):
            a1 = _lrelu(as_o1 + ad_j)
            a2 = _lrelu(as_o2 + ad_j)
            at = _lrelu(jnp.roll(as_j, 1, axis=0) + ad_j)
            at = jnp.where(t0_mask, _NEG, at)
            a0 = _lrelu(as_j + ad_j)
            m = jnp.maximum(jnp.maximum(a1, a2), jnp.maximum(at, a0))
            e1 = jnp.exp(a1 - m)
            e2 = jnp.exp(a2 - m)
            et = jnp.exp(at - m)
            e0 = jnp.exp(a0 - m)
            r = 1.0 / (e1 + e2 + et + e0 + 1e-16)
            h_prev = jnp.roll(h_j, 1, axis=0)
            return combine([(e1 * r, h_o1), (e2 * r, h_o2),
                            (et * r, h_prev), (e0 * r, h_j)])

        oM = face(hM, asM, adM, hL, asL, hR, asR)
        oL = face(hL, asL, adL, hM, asM, hR, asR)
        oR = face(hR, asR, adR, hM, asM, hL, asL)

        aL = _lrelu(asL + adA)
        aR = _lrelu(asR + adA)
        a0 = _lrelu(asA + adA)
        m = jnp.maximum(jnp.maximum(aL, aR), a0)
        eL = jnp.exp(aL - m)
        eR = jnp.exp(aR - m)
        e0 = jnp.exp(a0 - m)
        r = 1.0 / (eL + eR + e0 + 1e-16)
        oA = combine([(eL * r, hL), (eR * r, hR), (e0 * r, hA)])

        bias = b_ref[...]
        if concat:
            return tuple(jnp.concatenate(o, axis=1) + bias for o in (oM, oL, oR, oA))
        return tuple(0.25 * (o[0] + o[1] + o[2] + o[3]) + bias for o in (oM, oL, oR, oA))

    xM, xL, xR, xA = gat(xM, xL, xR, xA, W0_ref, as0_ref, ad0_ref, b0_ref, True)
    xM, xL, xR, xA = gat(xM, xL, xR, xA, W1_ref, as1_ref, ad1_ref, b1_ref, True)
    xM, xL, xR, xA = gat(xM, xL, xR, xA, W2_ref, as2_ref, ad2_ref, b2_ref, False)

    gamma = g_ref[...]
    beta = be_ref[...]
    for r_idx, x in enumerate((xM, xL, xR, xA)):
        mu = jnp.mean(x, axis=1, keepdims=True)
        var = jnp.mean((x - mu) ** 2, axis=1, keepdims=True)
        xn = (x - mu) * jax.lax.rsqrt(var + 1e-5) * gamma + beta
        sums_ref[0, r_idx, :] = jnp.sum(xn, axis=0)


def kernel(region_mouth, region_left_eye, region_right_eye, audio_embeddings,
           W_mouth, b_mouth, W_leye, b_leye, W_reye, b_reye, W_audio, b_audio,
           W0, att_src0, att_dst0, bias0, W1, att_src1, att_dst1, bias1,
           W2, att_src2, att_dst2, bias2, ln_gamma, ln_beta):
    f32 = jnp.float32

    row = lambda v: v.reshape(1, -1)

    full = lambda shape: pl.BlockSpec(shape, lambda b: (0,) * len(shape))
    per_b = lambda shape: pl.BlockSpec(shape, lambda b: (b, 0, 0))

    sums = pl.pallas_call(
        _fused_kernel,
        grid=(B,),
        in_specs=[
            per_b((1, T, 512)), per_b((1, T, 256)), per_b((1, T, 256)), per_b((1, T_A, 128)),
            full((512, H)), full((1, H)),
            full((256, H)), full((1, H)),
            full((256, H)), full((1, H)),
            full((128, H)), full((1, H)),
            full((H, HH)), full((HEADS, H)), full((HEADS, H)), full((1, HH)),
            full((HH, HH)), full((HEADS, H)), full((HEADS, H)), full((1, HH)),
            full((HH, HH)), full((HEADS, H)), full((HEADS, H)), full((1, H)),
            full((1, H)), full((1, H)),
        ],
        out_specs=pl.BlockSpec((1, 4, H), lambda b: (b, 0, 0)),
        out_shape=jax.ShapeDtypeStruct((B, 4, H), f32),
        compiler_params=pltpu.CompilerParams(
            dimension_semantics=("arbitrary",),
        ),
    )(region_mouth, region_left_eye, region_right_eye, audio_embeddings,
      W_mouth, row(b_mouth), W_leye, row(b_leye), W_reye, row(b_reye),
      W_audio, row(b_audio),
      W0, att_src0, att_dst0, row(bias0),
      W1, att_src1, att_dst1, row(bias1),
      W2, att_src2, att_dst2, row(bias2),
      row(ln_gamma), row(ln_beta))

    s2 = sums.reshape(2, 4, 4, H)
    out = s2.sum(axis=1).transpose(1, 0, 2).reshape(B, H)
    return out / 2048.0


# speedup vs baseline: 1.1733x; 1.0143x over previous
import jax
import jax.numpy as jnp
from jax.experimental import pallas as pl
from jax.experimental.pallas import tpu as pltpu

B = 8
T = 512
T_A = 512
H = 256
HEADS = 4
HH = HEADS * H

_NEG = -1e30


def _lrelu(x):
    return jnp.maximum(x, 0.2 * x)


def _fused_kernel(rm_ref, rl_ref, rr_ref, ra_ref,
                  Wm_ref, bm_ref, Wl_ref, bl_ref, Wr_ref, br_ref, Wa_ref, ba_ref,
                  W0_ref, as0_ref, ad0_ref, b0_ref,
                  W1_ref, as1_ref, ad1_ref, b1_ref,
                  W2_ref, as2_ref, ad2_ref, b2_ref,
                  g_ref, be_ref,
                  sums_ref):
    f32 = jnp.float32

    def mm(a, w):
        return jnp.dot(a, w, preferred_element_type=f32)

    xM = mm(rm_ref[0], Wm_ref[...]) + bm_ref[...]
    xL = mm(rl_ref[0], Wl_ref[...]) + bl_ref[...]
    xR = mm(rr_ref[0], Wr_ref[...]) + br_ref[...]
    xA = mm(ra_ref[0], Wa_ref[...]) + ba_ref[...]

    t0_mask = jax.lax.broadcasted_iota(jnp.int32, (T, HEADS), 0) == 0
    hsl = [slice(k * H, (k + 1) * H) for k in range(HEADS)]

    def gat(xM, xL, xR, xA, W_ref, as_ref, ad_ref, b_ref, concat):
        W = W_ref[...]
        att_s = as_ref[...]
        att_d = ad_ref[...]

        def logits(h):
            cols_s, cols_d = [], []
            for k in range(HEADS):
                hk = h[:, hsl[k]]
                cols_s.append(jnp.sum(hk * att_s[k:k + 1, :], axis=1, keepdims=True))
                cols_d.append(jnp.sum(hk * att_d[k:k + 1, :], axis=1, keepdims=True))
            return jnp.concatenate(cols_s, axis=1), jnp.concatenate(cols_d, axis=1)

        hM, hL, hR, hA = mm(xM, W), mm(xL, W), mm(xR, W), mm(xA, W)
        asM, adM = logits(hM)
        asL, adL = logits(hL)
        asR, adR = logits(hR)
        asA, adA = logits(hA)

        def combine(ws_hs):
            return [sum(w[:, k:k + 1] * h[:, hsl[k]] for w, h in ws_hs)
                    for k in range(HEADS)]

        def face(h_j, as_j, ad_j, h_o1, as_o1, h_o2, as_o2):
            a1 = _lrelu(as_o1 + ad_j)
            a2 = _lrelu(as_o2 + ad_j)
            at = _lrelu(jnp.roll(as_j, 1, axis=0) + ad_j)
            at = jnp.where(t0_mask, _NEG, at)
            a0 = _lrelu(as_j + ad_j)
            m = jnp.maximum(jnp.maximum(a1, a2), jnp.maximum(at, a0))
            e1 = jnp.exp(a1 - m)
            e2 = jnp.exp(a2 - m)
            et = jnp.exp(at - m)
            e0 = jnp.exp(a0 - m)
            r = 1.0 / (e1 + e2 + et + e0 + 1e-16)
            h_prev = jnp.roll(h_j, 1, axis=0)
            return combine([(e1 * r, h_o1), (e2 * r, h_o2),
                            (et * r, h_prev), (e0 * r, h_j)])

        oM = face(hM, asM, adM, hL, asL, hR, asR)
        oL = face(hL, asL, adL, hM, asM, hR, asR)
        oR = face(hR, asR, adR, hM, asM, hL, asL)

        aL = _lrelu(asL + adA)
        aR = _lrelu(asR + adA)
        a0 = _lrelu(asA + adA)
        m = jnp.maximum(jnp.maximum(aL, aR), a0)
        eL = jnp.exp(aL - m)
        eR = jnp.exp(aR - m)
        e0 = jnp.exp(a0 - m)
        r = 1.0 / (eL + eR + e0 + 1e-16)
        oA = combine([(eL * r, hL), (eR * r, hR), (e0 * r, hA)])

        bias = b_ref[...]
        if concat:
            return tuple(jnp.concatenate(o, axis=1) + bias for o in (oM, oL, oR, oA))
        return tuple(0.25 * (o[0] + o[1] + o[2] + o[3]) + bias for o in (oM, oL, oR, oA))

    xM, xL, xR, xA = gat(xM, xL, xR, xA, W0_ref, as0_ref, ad0_ref, b0_ref, True)
    xM, xL, xR, xA = gat(xM, xL, xR, xA, W1_ref, as1_ref, ad1_ref, b1_ref, True)
    xM, xL, xR, xA = gat(xM, xL, xR, xA, W2_ref, as2_ref, ad2_ref, b2_ref, False)

    gamma = g_ref[...]
    beta = be_ref[...]
    for r_idx, x in enumerate((xM, xL, xR, xA)):
        mu = jnp.mean(x, axis=1, keepdims=True)
        var = jnp.mean((x - mu) ** 2, axis=1, keepdims=True)
        xn = (x - mu) * jax.lax.rsqrt(var + 1e-5) * gamma + beta
        sums_ref[0, r_idx, :] = jnp.sum(xn, axis=0)


def kernel(region_mouth, region_left_eye, region_right_eye, audio_embeddings,
           W_mouth, b_mouth, W_leye, b_leye, W_reye, b_reye, W_audio, b_audio,
           W0, att_src0, att_dst0, bias0, W1, att_src1, att_dst1, bias1,
           W2, att_src2, att_dst2, bias2, ln_gamma, ln_beta):
    f32 = jnp.float32

    row = lambda v: v.reshape(1, -1)

    full = lambda shape: pl.BlockSpec(shape, lambda b: (0,) * len(shape))
    per_b = lambda shape: pl.BlockSpec(shape, lambda b: (b, 0, 0))

    sums = pl.pallas_call(
        _fused_kernel,
        grid=(B,),
        in_specs=[
            per_b((1, T, 512)), per_b((1, T, 256)), per_b((1, T, 256)), per_b((1, T_A, 128)),
            full((512, H)), full((1, H)),
            full((256, H)), full((1, H)),
            full((256, H)), full((1, H)),
            full((128, H)), full((1, H)),
            full((H, HH)), full((HEADS, H)), full((HEADS, H)), full((1, HH)),
            full((HH, HH)), full((HEADS, H)), full((HEADS, H)), full((1, HH)),
            full((HH, HH)), full((HEADS, H)), full((HEADS, H)), full((1, H)),
            full((1, H)), full((1, H)),
        ],
        out_specs=pl.BlockSpec((1, 4, H), lambda b: (b, 0, 0)),
        out_shape=jax.ShapeDtypeStruct((B, 4, H), f32),
        compiler_params=pltpu.CompilerParams(
            dimension_semantics=("arbitrary",),
        ),
    )(region_mouth, region_left_eye, region_right_eye, audio_embeddings,
      W_mouth, row(b_mouth), W_leye, row(b_leye), W_reye, row(b_reye),
      W_audio, row(b_audio),
      W0, att_src0, att_dst0, row(bias0),
      W1, att_src1, att_dst1, row(bias1),
      W2, att_src2, att_dst2, row(bias2),
      row(ln_gamma), row(ln_beta))

    s2 = sums.reshape(2, 4, 4, H)
    out = s2.sum(axis=1).transpose(1, 0, 2).reshape(B, H)
    return out / 2048.0
